# baseline (speedup 1.0000x reference)
import functools

import numpy as np
import jax
import jax.numpy as jnp
from jax import lax
from jax.experimental import pallas as pl
from jax.experimental.pallas import tpu as pltpu
from jax.experimental.pallas import tpu_sc as plsc

_IMAGE = 200
_PC = 50
_NPTS = _PC * _PC
_LANES = 16
_PAD = 2560
_NCHUNK = _PAD // _LANES
_NC, _NS = 2, 16
_NW = _NC * _NS
_SC_FRAMES = 64


def _grid_geometry():
    x1 = np.linspace(0.0, 1.0, _PC).astype(np.float32)
    g = x1 * np.float32(_IMAGE - 1)
    i0 = np.clip(np.floor(g).astype(np.int32), 0, _IMAGE - 1)
    i1 = np.clip(i0 + 1, 0, _IMAGE - 1)
    w = g - i0.astype(np.float32)
    return i0, i1, w


def _build_interp_matrices():
    i0, i1, w = _grid_geometry()
    m = np.zeros((_PC, _IMAGE), np.float32)
    m[np.arange(_PC), i0] += 1.0 - w
    m[np.arange(_PC), i1] += w
    return m


_INTERP_W = _build_interp_matrices()


def _make_sc_sampler(n_frames):
    frames_per_w = n_frames // _NW
    mesh = plsc.VectorSubcoreMesh(core_axis_name="c", subcore_axis_name="s")

    @functools.partial(
        pl.kernel,
        mesh=mesh,
        out_type=jax.ShapeDtypeStruct((n_frames, _PAD), jnp.float32),
        compiler_params=pltpu.CompilerParams(needs_layout_passes=False),
        scratch_types=[
            pltpu.VMEM((_IMAGE, _IMAGE), jnp.float32),
            pltpu.VMEM((_IMAGE, _IMAGE), jnp.float32),
            pltpu.VMEM((_PAD,), jnp.float32),
            pltpu.SemaphoreType.DMA,
            pltpu.SemaphoreType.DMA,
        ],
    )
    def sampler(frames_hbm, out_hbm, fbuf0, fbuf1, ovec, sem0, sem1):
        wid = lax.axis_index("s") * _NC + lax.axis_index("c")
        base = wid * frames_per_w
        bufs = (fbuf0, fbuf1)
        sems = (sem0, sem1)
        lane = lax.iota(jnp.int32, _LANES)
        nxt = pltpu.async_copy(frames_hbm.at[base], fbuf0, sem0)
        for k in range(frames_per_w):
            cur_buf = bufs[k % 2]
            cur_cp = nxt
            if k + 1 < frames_per_w:
                nxt = pltpu.async_copy(
                    frames_hbm.at[base + k + 1], bufs[(k + 1) % 2],
                    sems[(k + 1) % 2])
            cur_cp.wait()

            @plsc.parallel_loop(0, _NCHUNK, unroll=4)
            def _chunk(i):
                p = i * _LANES + lane
                py = (p * 1311) >> 16
                px = p - py * _PC
                gx = (px.astype(jnp.float32) / np.float32(_PC - 1)
                      * np.float32(_IMAGE - 1))
                gy = (py.astype(jnp.float32) / np.float32(_PC - 1)
                      * np.float32(_IMAGE - 1))
                ix0 = jnp.minimum(gx.astype(jnp.int32), _IMAGE - 1)
                iy0 = jnp.minimum(gy.astype(jnp.int32), _IMAGE - 1)
                ix1 = jnp.minimum(ix0 + 1, _IMAGE - 1)
                iy1 = jnp.minimum(iy0 + 1, _IMAGE - 1)
                wx = gx - ix0.astype(jnp.float32)
                wy = gy - iy0.astype(jnp.float32)
                v0 = plsc.load_gather(cur_buf, [iy0, ix0])
                v1 = plsc.load_gather(cur_buf, [iy0, ix1])
                v2 = plsc.load_gather(cur_buf, [iy1, ix0])
                v3 = plsc.load_gather(cur_buf, [iy1, ix1])
                ovec[pl.ds(i * _LANES, _LANES)] = (
                    v0 * (1.0 - wx) * (1.0 - wy) + v1 * wx * (1.0 - wy)
                    + v2 * (1.0 - wx) * wy + v3 * wx * wy)

            pltpu.sync_copy(ovec, out_hbm.at[base + k])

    return sampler


_TC_BLK = 16


def _tc_matmul_half(frames, start, count):
    wy = jnp.asarray(_INTERP_W)
    wxt = jnp.asarray(_INTERP_W.T)

    def body(frame_ref, wy_ref, wxt_ref, out_ref):
        rm = frame_ref[...].reshape(_TC_BLK * _IMAGE, _IMAGE)
        tmp = jnp.dot(rm, wxt_ref[...], preferred_element_type=jnp.float32)
        tmp = tmp.reshape(_TC_BLK, _IMAGE, _PC)
        for f in range(_TC_BLK):
            out_ref[f] = jnp.dot(wy_ref[...], tmp[f],
                                 preferred_element_type=jnp.float32)

    blocks = count // _TC_BLK
    return pl.pallas_call(
        body,
        grid=(blocks,),
        in_specs=[
            pl.BlockSpec((_TC_BLK, _IMAGE, _IMAGE),
                         lambda i: (start // _TC_BLK + i, 0, 0)),
            pl.BlockSpec((_PC, _IMAGE), lambda i: (0, 0)),
            pl.BlockSpec((_IMAGE, _PC), lambda i: (0, 0)),
        ],
        out_specs=pl.BlockSpec((_TC_BLK, _PC, _PC), lambda i: (i, 0, 0)),
        out_shape=jax.ShapeDtypeStruct((count, _PC, _PC), jnp.float32),
        compiler_params=pltpu.CompilerParams(
            dimension_semantics=("arbitrary",)),
    )(frames, wy, wxt)


def kernel(input):
    B, T, C, H, W = input.shape
    n_frames = B * T * C
    frames = input.reshape(n_frames, H, W)
    sc_n = min(_SC_FRAMES, n_frames)
    sc_out = _make_sc_sampler(sc_n)(frames)
    parts = [sc_out[:, :_NPTS]]
    if n_frames > sc_n:
        tc_out = _tc_matmul_half(frames, sc_n, n_frames - sc_n)
        parts.append(tc_out.reshape(n_frames - sc_n, _NPTS))
    res = jnp.concatenate(parts, axis=0).reshape(B, T, C, _NPTS)
    return (res, res)


# speedup vs baseline: 1.0520x; 1.0520x over previous
import functools

import numpy as np
import jax
import jax.numpy as jnp
from jax import lax
from jax.experimental import pallas as pl
from jax.experimental.pallas import tpu as pltpu
from jax.experimental.pallas import tpu_sc as plsc

_IMAGE = 200
_PC = 50
_NPTS = _PC * _PC
_LANES = 16
_PAD = 2560
_NCHUNK = _PAD // _LANES
_NC, _NS = 2, 16
_NW = _NC * _NS
_SC_FRAMES = 96


def _grid_geometry():
    x1 = np.linspace(0.0, 1.0, _PC).astype(np.float32)
    g = x1 * np.float32(_IMAGE - 1)
    i0 = np.clip(np.floor(g).astype(np.int32), 0, _IMAGE - 1)
    i1 = np.clip(i0 + 1, 0, _IMAGE - 1)
    w = g - i0.astype(np.float32)
    return i0, i1, w


def _build_interp_matrices():
    i0, i1, w = _grid_geometry()
    m = np.zeros((_PC, _IMAGE), np.float32)
    m[np.arange(_PC), i0] += 1.0 - w
    m[np.arange(_PC), i1] += w
    return m


_INTERP_W = _build_interp_matrices()


def _make_sc_sampler(n_frames):
    frames_per_w = n_frames // _NW
    mesh = plsc.VectorSubcoreMesh(core_axis_name="c", subcore_axis_name="s")

    @functools.partial(
        pl.kernel,
        mesh=mesh,
        out_type=jax.ShapeDtypeStruct((n_frames, _PAD), jnp.float32),
        compiler_params=pltpu.CompilerParams(needs_layout_passes=False),
        scratch_types=[
            pltpu.VMEM((_IMAGE, _IMAGE), jnp.float32),
            pltpu.VMEM((_IMAGE, _IMAGE), jnp.float32),
            pltpu.VMEM((_PAD,), jnp.float32),
            pltpu.SemaphoreType.DMA,
            pltpu.SemaphoreType.DMA,
        ],
    )
    def sampler(frames_hbm, out_hbm, fbuf0, fbuf1, ovec, sem0, sem1):
        wid = lax.axis_index("s") * _NC + lax.axis_index("c")
        base = wid * frames_per_w
        bufs = (fbuf0, fbuf1)
        sems = (sem0, sem1)
        lane = lax.iota(jnp.int32, _LANES)
        nxt = pltpu.async_copy(frames_hbm.at[base], fbuf0, sem0)
        for k in range(frames_per_w):
            cur_buf = bufs[k % 2]
            cur_cp = nxt
            if k + 1 < frames_per_w:
                nxt = pltpu.async_copy(
                    frames_hbm.at[base + k + 1], bufs[(k + 1) % 2],
                    sems[(k + 1) % 2])
            cur_cp.wait()

            @plsc.parallel_loop(0, _NCHUNK, unroll=4)
            def _chunk(i):
                p = i * _LANES + lane
                py = (p * 1311) >> 16
                px = p - py * _PC
                gx = (px.astype(jnp.float32) / np.float32(_PC - 1)
                      * np.float32(_IMAGE - 1))
                gy = (py.astype(jnp.float32) / np.float32(_PC - 1)
                      * np.float32(_IMAGE - 1))
                ix0 = jnp.minimum(gx.astype(jnp.int32), _IMAGE - 1)
                iy0 = jnp.minimum(gy.astype(jnp.int32), _IMAGE - 1)
                ix1 = jnp.minimum(ix0 + 1, _IMAGE - 1)
                iy1 = jnp.minimum(iy0 + 1, _IMAGE - 1)
                wx = gx - ix0.astype(jnp.float32)
                wy = gy - iy0.astype(jnp.float32)
                v0 = plsc.load_gather(cur_buf, [iy0, ix0])
                v1 = plsc.load_gather(cur_buf, [iy0, ix1])
                v2 = plsc.load_gather(cur_buf, [iy1, ix0])
                v3 = plsc.load_gather(cur_buf, [iy1, ix1])
                ovec[pl.ds(i * _LANES, _LANES)] = (
                    v0 * (1.0 - wx) * (1.0 - wy) + v1 * wx * (1.0 - wy)
                    + v2 * (1.0 - wx) * wy + v3 * wx * wy)

            pltpu.sync_copy(ovec, out_hbm.at[base + k])

    return sampler


_TC_BLK = 32


def _tc_matmul_half(frames, start, count):
    wy = jnp.asarray(_INTERP_W)
    wxt = jnp.asarray(_INTERP_W.T)

    def body(frame_ref, wy_ref, wxt_ref, out_ref):
        rm = frame_ref[...].reshape(_TC_BLK * _IMAGE, _IMAGE)
        tmp = jnp.dot(rm, wxt_ref[...], preferred_element_type=jnp.float32)
        tmp = tmp.reshape(_TC_BLK, _IMAGE, _PC)
        for f in range(_TC_BLK):
            out_ref[f] = jnp.dot(wy_ref[...], tmp[f],
                                 preferred_element_type=jnp.float32)

    blocks = count // _TC_BLK
    return pl.pallas_call(
        body,
        grid=(blocks,),
        in_specs=[
            pl.BlockSpec((_TC_BLK, _IMAGE, _IMAGE),
                         lambda i: (start // _TC_BLK + i, 0, 0)),
            pl.BlockSpec((_PC, _IMAGE), lambda i: (0, 0)),
            pl.BlockSpec((_IMAGE, _PC), lambda i: (0, 0)),
        ],
        out_specs=pl.BlockSpec((_TC_BLK, _PC, _PC), lambda i: (i, 0, 0)),
        out_shape=jax.ShapeDtypeStruct((count, _PC, _PC), jnp.float32),
        compiler_params=pltpu.CompilerParams(
            dimension_semantics=("arbitrary",)),
    )(frames, wy, wxt)


def kernel(input):
    B, T, C, H, W = input.shape
    n_frames = B * T * C
    frames = input.reshape(n_frames, H, W)
    sc_n = min(_SC_FRAMES, n_frames)
    sc_out = _make_sc_sampler(sc_n)(frames)
    parts = [sc_out[:, :_NPTS]]
    if n_frames > sc_n:
        tc_out = _tc_matmul_half(frames, sc_n, n_frames - sc_n)
        parts.append(tc_out.reshape(n_frames - sc_n, _NPTS))
    res = jnp.concatenate(parts, axis=0).reshape(B, T, C, _NPTS)
    return (res, res)
